# Initial kernel scaffold; baseline (speedup 1.0000x reference)
import jax
import jax.numpy as jnp
from jax.experimental import pallas as pl


def kernel(times, data, t):
    raise NotImplementedError("write your pallas kernel here")


# speedup vs baseline: 1.3844x; 1.3844x over previous
import functools

import jax
import jax.numpy as jnp
from jax import lax
from jax.experimental import pallas as pl
from jax.experimental.pallas import tpu as pltpu
from jax.experimental.pallas import tpu_sc as plsc

L = 16
NC = 2
NS = 16
NW = NC * NS
IDXW = 128


def _interp_body(ntime, nbatch, chunk,
                 times_hbm, data_hbm, t_hbm, cidx_hbm, out_hbm,
                 idx_v, tcolv, t0v, tv, d0v, d1v, x0v, x1v, ov, sem):
    wid = lax.axis_index("s") * NC + lax.axis_index("c")
    base = wid * chunk

    cp_t0 = pltpu.make_async_copy(t_hbm.at[pl.ds(0, L)], t0v, sem)
    cp_t = pltpu.make_async_copy(t_hbm.at[pl.ds(base, chunk)], tv, sem)
    cp_i = pltpu.make_async_copy(cidx_hbm, idx_v, sem)
    cp_t0.start()
    cp_t.start()
    cp_i.start()
    cp_t0.wait()
    cp_t.wait()
    cp_i.wait()

    nseg = ntime // IDXW
    gathers = [
        pltpu.make_async_copy(
            times_hbm.at[idx_v.at[j]], tcolv.at[pl.ds(j * IDXW, IDXW)], sem)
        for j in range(nseg)
    ]
    for g in gathers:
        g.start()
    for g in gathers:
        g.wait()

    t0 = t0v[...][0]
    t0_vec = jnp.full((L,), t0, dtype=jnp.float32)

    cnt = jnp.zeros((L,), dtype=jnp.int32)
    for k in range(ntime // L):
        vals = tcolv[pl.ds(k * L, L)]
        cnt = cnt + jnp.where(vals < t0_vec, 1, 0).astype(jnp.int32)
    total = cnt[0]
    for j in range(1, L):
        total = total + cnt[j]
    gi = jnp.where(total >= ntime, 1, jnp.maximum(total, 1)).astype(jnp.int32)
    gim1 = gi - 1

    off0 = pl.multiple_of(gim1 * nbatch + base, 8)
    off1 = pl.multiple_of(gi * nbatch + base, 8)
    cp_d0 = pltpu.make_async_copy(data_hbm.at[pl.ds(off0, chunk)], d0v, sem)
    cp_d1 = pltpu.make_async_copy(data_hbm.at[pl.ds(off1, chunk)], d1v, sem)
    cp_x0 = pltpu.make_async_copy(times_hbm.at[pl.ds(off0, chunk)], x0v, sem)
    cp_x1 = pltpu.make_async_copy(times_hbm.at[pl.ds(off1, chunk)], x1v, sem)
    cp_d0.start()
    cp_d1.start()
    cp_x0.start()
    cp_x1.start()
    cp_d0.wait()
    cp_d1.wait()
    cp_x0.wait()
    cp_x1.wait()

    for k in range(chunk // L):
        s = pl.ds(k * L, L)
        a = d0v[s]
        b = d1v[s]
        p = x0v[s]
        q = x1v[s]
        tt = tv[s]
        ov[s] = a + (b - a) / (q - p) * (tt - p)

    pltpu.sync_copy(ov, out_hbm.at[pl.ds(base, chunk)])


def kernel(times, data, t):
    ntime, nbatch = times.shape
    chunk = nbatch // NW
    times_flat = times.reshape(-1)
    data_flat = data.reshape(-1)
    cidx = (jnp.arange(ntime, dtype=jnp.int32) * nbatch).reshape(
        ntime // IDXW, IDXW)
    mesh = plsc.VectorSubcoreMesh(core_axis_name="c", subcore_axis_name="s")
    body = functools.partial(_interp_body, ntime, nbatch, chunk)
    run = pl.kernel(
        body,
        mesh=mesh,
        out_type=jax.ShapeDtypeStruct((nbatch,), jnp.float32),
        scratch_types=[
            pltpu.VMEM((ntime // IDXW, IDXW), jnp.int32),
            pltpu.VMEM((ntime,), jnp.float32),
            pltpu.VMEM((L,), jnp.float32),
            pltpu.VMEM((chunk,), jnp.float32),
            pltpu.VMEM((chunk,), jnp.float32),
            pltpu.VMEM((chunk,), jnp.float32),
            pltpu.VMEM((chunk,), jnp.float32),
            pltpu.VMEM((chunk,), jnp.float32),
            pltpu.VMEM((chunk,), jnp.float32),
            pltpu.SemaphoreType.DMA,
        ],
    )
    return run(times_flat, data_flat, t, cidx)


# speedup vs baseline: 7.4066x; 5.3499x over previous
import functools

import jax
import jax.numpy as jnp
from jax import lax
from jax.experimental import pallas as pl
from jax.experimental.pallas import tpu as pltpu
from jax.experimental.pallas import tpu_sc as plsc

L = 16
NC = 2
NS = 16
NW = NC * NS
CB = 128


def _interp_body(ntime, nbatch, chunk,
                 times_hbm, data_hbm, t_hbm, out_hbm,
                 colblk, t0v, tv, dblk, xblk, ov, cnt_smem, sem):
    cid = lax.axis_index("c")
    sid = lax.axis_index("s")
    wid = sid * NC + cid
    base = pl.multiple_of(wid * chunk, 128)
    rows_per_tile = ntime // NS

    row0 = pl.multiple_of(sid * rows_per_tile, 8)
    cp_t0 = pltpu.make_async_copy(t_hbm.at[pl.ds(0, L)], t0v, sem)
    cp_t = pltpu.make_async_copy(t_hbm.at[pl.ds(base, chunk)], tv, sem)
    cp_c = pltpu.make_async_copy(
        times_hbm.at[pl.ds(row0, rows_per_tile), pl.ds(0, CB)], colblk, sem)
    cp_t0.start()
    cp_t.start()
    cp_c.start()
    cp_t0.wait()
    cp_c.wait()

    t0 = t0v[...][0]
    t0_vec = jnp.full((L,), t0, dtype=jnp.float32)

    cnt = jnp.zeros((L,), dtype=jnp.int32)
    for r in range(rows_per_tile):
        vals = colblk[r, pl.ds(0, L)]
        cnt = cnt + jnp.where(vals < t0_vec, 1, 0).astype(jnp.int32)

    @pl.when(sid == 0)
    def _zero():
        cnt_smem[0] = 0

    plsc.subcore_barrier()
    plsc.fetch_and_add(cnt_smem.at[0], cnt[0], subcore_id=0)
    plsc.subcore_barrier()
    total = plsc.fetch_and_add(cnt_smem.at[0], 0, subcore_id=0)

    gi = jnp.where(total >= ntime, 1, jnp.maximum(total, 1)).astype(jnp.int32)
    gim1 = gi - 1

    rb = pl.multiple_of(
        jnp.minimum((gim1 // 8) * 8, ntime - 2 * 8).astype(jnp.int32), 8)
    loff = gim1 - rb
    cp_d = pltpu.make_async_copy(
        data_hbm.at[pl.ds(rb, 2 * 8), pl.ds(base, chunk)], dblk, sem)
    cp_x = pltpu.make_async_copy(
        times_hbm.at[pl.ds(rb, 2 * 8), pl.ds(base, chunk)], xblk, sem)
    cp_d.start()
    cp_x.start()
    cp_d.wait()
    cp_x.wait()
    cp_t.wait()

    for k in range(chunk // L):
        s = pl.ds(k * L, L)
        a = dblk[loff, s]
        b = dblk[loff + 1, s]
        p = xblk[loff, s]
        q = xblk[loff + 1, s]
        tt = tv[s]
        ov[s] = a + (b - a) / (q - p) * (tt - p)

    pltpu.sync_copy(ov, out_hbm.at[pl.ds(base, chunk)])


def kernel(times, data, t):
    ntime, nbatch = times.shape
    chunk = nbatch // NW
    mesh = plsc.VectorSubcoreMesh(core_axis_name="c", subcore_axis_name="s")
    body = functools.partial(_interp_body, ntime, nbatch, chunk)
    run = pl.kernel(
        body,
        mesh=mesh,
        out_type=jax.ShapeDtypeStruct((nbatch,), jnp.float32),
        scratch_types=[
            pltpu.VMEM((ntime // NS, CB), jnp.float32),
            pltpu.VMEM((L,), jnp.float32),
            pltpu.VMEM((chunk,), jnp.float32),
            pltpu.VMEM((2 * 8, chunk), jnp.float32),
            pltpu.VMEM((2 * 8, chunk), jnp.float32),
            pltpu.VMEM((chunk,), jnp.float32),
            pltpu.SMEM((1,), jnp.int32),
            pltpu.SemaphoreType.DMA,
        ],
    )
    return run(times, data, t)


# speedup vs baseline: 7.7154x; 1.0417x over previous
import functools

import jax
import jax.numpy as jnp
from jax import lax
from jax.experimental import pallas as pl
from jax.experimental.pallas import tpu as pltpu
from jax.experimental.pallas import tpu_sc as plsc

L = 16
NC = 2
NS = 16
NW = NC * NS
CB = 128


def _interp_body(ntime, nbatch, chunk,
                 times_hbm, data_hbm, t_hbm, out_hbm,
                 colblk, t0v, tv, dblk, xblk, ov, cnt_smem, sem):
    cid = lax.axis_index("c")
    sid = lax.axis_index("s")
    wid = sid * NC + cid
    base = pl.multiple_of(wid * chunk, 128)
    rows_per_tile = ntime // NS

    row0 = pl.multiple_of(sid * rows_per_tile, 8)
    cp_t0 = pltpu.make_async_copy(t_hbm.at[pl.ds(0, L)], t0v, sem)
    cp_t = pltpu.make_async_copy(t_hbm.at[pl.ds(base, chunk)], tv, sem)
    cp_c = pltpu.make_async_copy(
        times_hbm.at[pl.ds(row0, rows_per_tile), pl.ds(0, CB)], colblk, sem)
    cp_d = pltpu.make_async_copy(
        data_hbm.at[pl.ds(0, 2 * 8), pl.ds(base, chunk)], dblk, sem)
    cp_x = pltpu.make_async_copy(
        times_hbm.at[pl.ds(0, 2 * 8), pl.ds(base, chunk)], xblk, sem)
    cp_t0.start()
    cp_t.start()
    cp_c.start()
    cp_d.start()
    cp_x.start()
    cp_t0.wait()
    cp_c.wait()

    t0 = t0v[...][0]
    t0_vec = jnp.full((L,), t0, dtype=jnp.float32)

    cnt = jnp.zeros((L,), dtype=jnp.int32)
    for r in range(rows_per_tile):
        vals = colblk[r, pl.ds(0, L)]
        cnt = cnt + jnp.where(vals < t0_vec, 1, 0).astype(jnp.int32)

    @pl.when(sid == 0)
    def _zero():
        cnt_smem[0] = 0

    plsc.subcore_barrier()
    plsc.fetch_and_add(cnt_smem.at[0], cnt[0], subcore_id=0)
    plsc.subcore_barrier()
    total = plsc.fetch_and_add(cnt_smem.at[0], 0, subcore_id=0)

    gi = jnp.where(total >= ntime, 1, jnp.maximum(total, 1)).astype(jnp.int32)
    gim1 = gi - 1

    rb = pl.multiple_of(
        jnp.minimum((gim1 // 8) * 8, ntime - 2 * 8).astype(jnp.int32), 8)
    loff = gim1 - rb
    cp_d.wait()
    cp_x.wait()

    @pl.when(rb != 0)
    def _refetch():
        cp_d2 = pltpu.make_async_copy(
            data_hbm.at[pl.ds(rb, 2 * 8), pl.ds(base, chunk)], dblk, sem)
        cp_x2 = pltpu.make_async_copy(
            times_hbm.at[pl.ds(rb, 2 * 8), pl.ds(base, chunk)], xblk, sem)
        cp_d2.start()
        cp_x2.start()
        cp_d2.wait()
        cp_x2.wait()

    cp_t.wait()

    for k in range(chunk // L):
        s = pl.ds(k * L, L)
        a = dblk[loff, s]
        b = dblk[loff + 1, s]
        p = xblk[loff, s]
        q = xblk[loff + 1, s]
        tt = tv[s]
        ov[s] = a + (b - a) / (q - p) * (tt - p)

    pltpu.sync_copy(ov, out_hbm.at[pl.ds(base, chunk)])


def kernel(times, data, t):
    ntime, nbatch = times.shape
    chunk = nbatch // NW
    mesh = plsc.VectorSubcoreMesh(core_axis_name="c", subcore_axis_name="s")
    body = functools.partial(_interp_body, ntime, nbatch, chunk)
    run = pl.kernel(
        body,
        mesh=mesh,
        out_type=jax.ShapeDtypeStruct((nbatch,), jnp.float32),
        scratch_types=[
            pltpu.VMEM((ntime // NS, CB), jnp.float32),
            pltpu.VMEM((L,), jnp.float32),
            pltpu.VMEM((chunk,), jnp.float32),
            pltpu.VMEM((2 * 8, chunk), jnp.float32),
            pltpu.VMEM((2 * 8, chunk), jnp.float32),
            pltpu.VMEM((chunk,), jnp.float32),
            pltpu.SMEM((1,), jnp.int32),
            pltpu.SemaphoreType.DMA,
        ],
    )
    return run(times, data, t)
